# baseline (speedup 1.0000x reference)
import functools

import jax
import jax.numpy as jnp
from jax import lax
from jax.experimental import pallas as pl
from jax.experimental.pallas import tpu as pltpu
from jax.experimental.pallas import tpu_sc as plsc


def _mlp_kernel(xc_ref, e_ref, b1t_ref, g_ref, b2t_ref, out_ref):
    bh = e_ref.shape[1]
    gw = g_ref.shape[0]
    hist = g_ref.shape[1] // gw
    n_groups = bh // gw

    h = jnp.maximum(
        jnp.dot(xc_ref[...], e_ref[...], preferred_element_type=jnp.float32)
        + b1t_ref[0][None, :],
        0.0,
    )

    for g in range(n_groups):
        og = jnp.dot(h[:, g * gw:(g + 1) * gw], g_ref[...],
                     preferred_element_type=jnp.float32)
        for t in range(hist):
            lo = t * bh + g * gw
            out_ref[:, lo:lo + gw] = (
                og[:, t * gw:(t + 1) * gw] + b2t_ref[0][None, lo:lo + gw]
            ).astype(jnp.bfloat16)


def _tc_mlp_bf16(input, W1, b1, W2, b2):
    B, N, in_dim = input.shape
    hid = W1.shape[1]
    hist = W2.shape[1] // hid

    nb = 400

    xc = jnp.transpose(input, (1, 2, 0)).reshape(N, in_dim * B)
    eye_b = jnp.eye(B, dtype=jnp.float32)
    e_mat = jnp.einsum('bc,dk->dbck', eye_b, W1).reshape(in_dim * B, B * hid)
    b1t = jnp.tile(b1, B).reshape(1, B * hid)
    w2r = W2.reshape(hid, hist, hid)
    eye4 = jnp.eye(4, dtype=jnp.float32)
    g_mat = jnp.einsum('bc,ktj->bktcj', eye4, w2r).reshape(4 * hid,
                                                           hist * 4 * hid)
    b2t = jnp.tile(b2.reshape(hist, 1, hid), (1, B, 1)).reshape(1,
                                                                hist * B * hid)

    return pl.pallas_call(
        _mlp_kernel,
        grid=(N // nb,),
        in_specs=[
            pl.BlockSpec((nb, in_dim * B), lambda i: (i, 0)),
            pl.BlockSpec((in_dim * B, B * hid), lambda i: (0, 0)),
            pl.BlockSpec((1, B * hid), lambda i: (0, 0)),
            pl.BlockSpec((4 * hid, hist * 4 * hid), lambda i: (0, 0)),
            pl.BlockSpec((1, hist * B * hid), lambda i: (0, 0)),
        ],
        out_specs=pl.BlockSpec((nb, hist * B * hid), lambda i: (i, 0)),
        out_shape=jax.ShapeDtypeStruct((N, hist * B * hid), jnp.bfloat16),
        compiler_params=pltpu.CompilerParams(
            dimension_semantics=("parallel",),
        ),
    )(xc, e_mat, b1t, g_mat, b2t)


_NW = 32
_WCHUNK = 32000
_UNROLL = 8


def _sc_widen_body(in_hbm, out_hbm, in_v, out_v):
    words = in_hbm.shape[0]
    per_w = words // _NW
    n_chunks = per_w // _WCHUNK
    groups = _WCHUNK // 16
    wid = lax.axis_index("s") * 2 + lax.axis_index("c")
    base = wid * per_w
    iota16 = lax.iota(jnp.int32, 16)
    himask = jnp.int32(-65536)

    def chunk_body(c, carry):
        off = base + c * _WCHUNK
        pltpu.sync_copy(in_hbm.at[pl.ds(off, _WCHUNK)], in_v)

        def grp_body(i, carry2):
            for u in range(_UNROLL):
                g = i * _UNROLL + u
                w32 = in_v[pl.ds(g * 16, 16)]
                evens = lax.shift_left(w32, jnp.int32(16))
                odds = lax.bitwise_and(w32, himask)
                eidx = g * 32 + 2 * iota16
                plsc.store_scatter(out_v, [eidx], evens)
                plsc.store_scatter(out_v, [eidx + 1], odds)
            return carry2

        lax.fori_loop(0, groups // _UNROLL, grp_body, 0)
        pltpu.sync_copy(out_v, out_hbm.at[pl.ds(off * 2, _WCHUNK * 2)])
        return carry

    lax.fori_loop(0, n_chunks, chunk_body, 0)


def _sc_widen(y16_flat):
    total = y16_flat.shape[0]
    y32 = lax.bitcast_convert_type(
        y16_flat.reshape(total // 2, 2), jnp.int32)
    mesh = plsc.VectorSubcoreMesh(core_axis_name="c", subcore_axis_name="s",
                                  num_cores=2, num_subcores=16)
    fn = functools.partial(
        pl.kernel,
        mesh=mesh,
        out_type=jax.ShapeDtypeStruct((total,), jnp.int32),
        scratch_types=[
            pltpu.VMEM((_WCHUNK,), jnp.int32),
            pltpu.VMEM((_WCHUNK * 2,), jnp.int32),
        ],
        compiler_params=pltpu.CompilerParams(
            needs_layout_passes=False,
            skip_device_barrier=True,
        ),
    )(_sc_widen_body)
    return lax.bitcast_convert_type(fn(y32), jnp.float32)


def kernel(input, W1, b1, W2, b2):
    B, N, in_dim = input.shape
    hid = W1.shape[1]
    hist = W2.shape[1] // hid

    y16 = _tc_mlp_bf16(input, W1, b1, W2, b2)
    out = _sc_widen(y16.reshape(N * hist * B * hid))
    return out.reshape(N, hist, B, hid)


# speedup vs baseline: 17.2700x; 17.2700x over previous
import functools

import jax
import jax.numpy as jnp
from jax import lax
from jax.experimental import pallas as pl
from jax.experimental.pallas import tpu as pltpu
from jax.experimental.pallas import tpu_sc as plsc


def _mlp_kernel(xc_ref, e_ref, b1t_ref, g_ref, b2t_ref, out_ref):
    bh = e_ref.shape[1]
    gw = g_ref.shape[0]
    hist = g_ref.shape[1] // gw
    n_groups = bh // gw

    h = jnp.maximum(
        jnp.dot(xc_ref[...], e_ref[...], preferred_element_type=jnp.float32)
        + b1t_ref[0][None, :],
        0.0,
    )

    rnd = jnp.int32(32768)
    himask = jnp.int32(-65536)

    for e in range(n_groups // 2):
        oa = jnp.dot(h[:, (2 * e) * gw:(2 * e + 1) * gw], g_ref[...],
                     preferred_element_type=jnp.float32)
        ob = jnp.dot(h[:, (2 * e + 1) * gw:(2 * e + 2) * gw], g_ref[...],
                     preferred_element_type=jnp.float32)
        for t in range(hist):
            la = t * bh + (2 * e) * gw
            lb = t * bh + (2 * e + 1) * gw
            a = oa[:, t * gw:(t + 1) * gw] + b2t_ref[0][None, la:la + gw]
            b = ob[:, t * gw:(t + 1) * gw] + b2t_ref[0][None, lb:lb + gw]
            ai = lax.bitcast_convert_type(a, jnp.int32) + rnd
            bi = lax.bitcast_convert_type(b, jnp.int32) + rnd
            w = lax.bitwise_or(
                lax.shift_right_logical(ai, jnp.int32(16)),
                lax.bitwise_and(bi, himask),
            )
            lo = (t * (n_groups // 2) + e) * gw
            out_ref[:, lo:lo + gw] = w


def _tc_mlp_bf16(input, W1, b1, W2, b2):
    B, N, in_dim = input.shape
    hid = W1.shape[1]
    hist = W2.shape[1] // hid

    nb = 400

    xc = jnp.transpose(input, (1, 2, 0)).reshape(N, in_dim * B)
    eye_b = jnp.eye(B, dtype=jnp.float32)
    e_mat = jnp.einsum('bc,dk->dbck', eye_b, W1).reshape(in_dim * B, B * hid)
    b1t = jnp.tile(b1, B).reshape(1, B * hid)
    w2r = W2.reshape(hid, hist, hid)
    eye4 = jnp.eye(4, dtype=jnp.float32)
    g_mat = jnp.einsum('bc,ktj->bktcj', eye4, w2r).reshape(4 * hid,
                                                           hist * 4 * hid)
    b2t = jnp.tile(b2.reshape(hist, 1, hid), (1, B, 1)).reshape(1,
                                                                hist * B * hid)

    return pl.pallas_call(
        _mlp_kernel,
        grid=(N // nb,),
        in_specs=[
            pl.BlockSpec((nb, in_dim * B), lambda i: (i, 0)),
            pl.BlockSpec((in_dim * B, B * hid), lambda i: (0, 0)),
            pl.BlockSpec((1, B * hid), lambda i: (0, 0)),
            pl.BlockSpec((4 * hid, hist * 4 * hid), lambda i: (0, 0)),
            pl.BlockSpec((1, hist * B * hid), lambda i: (0, 0)),
        ],
        out_specs=pl.BlockSpec((nb, hist * B * hid // 2), lambda i: (i, 0)),
        out_shape=jax.ShapeDtypeStruct((N, hist * B * hid // 2), jnp.int32),
        compiler_params=pltpu.CompilerParams(
            dimension_semantics=("parallel",),
        ),
    )(xc, e_mat, b1t, g_mat, b2t)


_NW = 32
_WCHUNK = 32000
_UNROLL = 8


def _sc_widen_body(in_hbm, out_hbm, in_v, out_v):
    words = in_hbm.shape[0]
    per_w = words // _NW
    n_chunks = per_w // _WCHUNK
    spans = _WCHUNK // 128
    wid = lax.axis_index("s") * 2 + lax.axis_index("c")
    base = wid * per_w
    himask = jnp.int32(-65536)

    def chunk_body(c, carry):
        off = base + c * _WCHUNK
        pltpu.sync_copy(in_hbm.at[pl.ds(off, _WCHUNK)], in_v)

        def span_body(s, carry2):
            for u in range(8):
                w32 = in_v[pl.ds(s * 128 + u * 16, 16)]
                out_v[pl.ds(s * 256 + u * 16, 16)] = (
                    lax.shift_left(w32, jnp.int32(16)))
                out_v[pl.ds(s * 256 + 128 + u * 16, 16)] = (
                    lax.bitwise_and(w32, himask))
            return carry2

        lax.fori_loop(0, spans, span_body, 0)
        pltpu.sync_copy(out_v, out_hbm.at[pl.ds(off * 2, _WCHUNK * 2)])
        return carry

    lax.fori_loop(0, n_chunks, chunk_body, 0)


def _sc_widen(y32):
    total = y32.shape[0] * 2
    mesh = plsc.VectorSubcoreMesh(core_axis_name="c", subcore_axis_name="s",
                                  num_cores=2, num_subcores=16)
    fn = functools.partial(
        pl.kernel,
        mesh=mesh,
        out_type=jax.ShapeDtypeStruct((total,), jnp.int32),
        scratch_types=[
            pltpu.VMEM((_WCHUNK,), jnp.int32),
            pltpu.VMEM((_WCHUNK * 2,), jnp.int32),
        ],
        compiler_params=pltpu.CompilerParams(
            needs_layout_passes=False,
            skip_device_barrier=True,
        ),
    )(_sc_widen_body)
    return lax.bitcast_convert_type(fn(y32), jnp.float32)


def kernel(input, W1, b1, W2, b2):
    B, N, in_dim = input.shape
    hid = W1.shape[1]
    hist = W2.shape[1] // hid

    y32 = _tc_mlp_bf16(input, W1, b1, W2, b2)
    out = _sc_widen(y32.reshape(N * hist * B * hid // 2))
    return out.reshape(N, hist, B, hid)


# speedup vs baseline: 38.3235x; 2.2191x over previous
import functools

import jax
import jax.numpy as jnp
from jax import lax
from jax.experimental import pallas as pl
from jax.experimental.pallas import tpu as pltpu
from jax.experimental.pallas import tpu_sc as plsc


def _mlp_kernel(xc_ref, e_ref, b1t_ref, g_ref, b2t_ref, out_ref):
    bh = e_ref.shape[1]
    gw = g_ref.shape[0]
    hist = g_ref.shape[1] // gw
    n_groups = bh // gw

    h = jnp.maximum(
        jnp.dot(xc_ref[...], e_ref[...], preferred_element_type=jnp.float32)
        + b1t_ref[0][None, :],
        0.0,
    )

    rnd = jnp.int32(32768)
    himask = jnp.int32(-65536)

    for e in range(n_groups // 2):
        oa = jnp.dot(h[:, (2 * e) * gw:(2 * e + 1) * gw], g_ref[...],
                     preferred_element_type=jnp.float32)
        ob = jnp.dot(h[:, (2 * e + 1) * gw:(2 * e + 2) * gw], g_ref[...],
                     preferred_element_type=jnp.float32)
        for t in range(hist):
            la = t * bh + (2 * e) * gw
            lb = t * bh + (2 * e + 1) * gw
            a = oa[:, t * gw:(t + 1) * gw] + b2t_ref[0][None, la:la + gw]
            b = ob[:, t * gw:(t + 1) * gw] + b2t_ref[0][None, lb:lb + gw]
            ai = lax.bitcast_convert_type(a, jnp.int32) + rnd
            bi = lax.bitcast_convert_type(b, jnp.int32) + rnd
            w = lax.bitwise_or(
                lax.shift_right_logical(ai, jnp.int32(16)),
                lax.bitwise_and(bi, himask),
            )
            lo = (t * (n_groups // 2) + e) * gw
            out_ref[:, lo:lo + gw] = w


def _tc_mlp_bf16(input, W1, b1, W2, b2):
    B, N, in_dim = input.shape
    hid = W1.shape[1]
    hist = W2.shape[1] // hid

    nb = 400

    xc = jnp.transpose(input, (1, 2, 0)).reshape(N, in_dim * B)
    eye_b = jnp.eye(B, dtype=jnp.float32)
    e_mat = jnp.einsum('bc,dk->dbck', eye_b, W1).reshape(in_dim * B, B * hid)
    b1t = jnp.tile(b1, B).reshape(1, B * hid)
    w2r = W2.reshape(hid, hist, hid)
    eye4 = jnp.eye(4, dtype=jnp.float32)
    g_mat = jnp.einsum('bc,ktj->bktcj', eye4, w2r).reshape(4 * hid,
                                                           hist * 4 * hid)
    b2t = jnp.tile(b2.reshape(hist, 1, hid), (1, B, 1)).reshape(1,
                                                                hist * B * hid)

    return pl.pallas_call(
        _mlp_kernel,
        grid=(N // nb,),
        in_specs=[
            pl.BlockSpec((nb, in_dim * B), lambda i: (i, 0)),
            pl.BlockSpec((in_dim * B, B * hid), lambda i: (0, 0)),
            pl.BlockSpec((1, B * hid), lambda i: (0, 0)),
            pl.BlockSpec((4 * hid, hist * 4 * hid), lambda i: (0, 0)),
            pl.BlockSpec((1, hist * B * hid), lambda i: (0, 0)),
        ],
        out_specs=pl.BlockSpec((nb, hist * B * hid // 2), lambda i: (i, 0)),
        out_shape=jax.ShapeDtypeStruct((N, hist * B * hid // 2), jnp.int32),
        compiler_params=pltpu.CompilerParams(
            dimension_semantics=("parallel",),
        ),
    )(xc, e_mat, b1t, g_mat, b2t)


_NW = 32
_ROWS = 8


def _sc_widen_body(in_hbm, out_hbm, in_v, out_v):
    n_rows = in_hbm.shape[0]
    wpr = in_hbm.shape[1]
    spans = wpr // 128
    n_chunks = n_rows // _ROWS
    wid = lax.axis_index("s") * 2 + lax.axis_index("c")
    c_lo = (wid * n_chunks) // _NW
    c_hi = ((wid + 1) * n_chunks) // _NW
    himask = jnp.int32(-65536)

    def chunk_body(c, carry):
        row0 = c * _ROWS
        pltpu.sync_copy(in_hbm.at[pl.ds(row0, _ROWS)], in_v)
        for r in range(_ROWS):
            def span_body(m, carry2, r=r):
                for j in range(8):
                    w32 = in_v[r, pl.ds(m * 128 + j * 16, 16)]
                    out_v[r, pl.ds(m * 256 + j * 16, 16)] = plsc.bitcast(
                        lax.shift_left(w32, jnp.int32(16)), jnp.float32)
                    out_v[r, pl.ds(m * 256 + 128 + j * 16, 16)] = plsc.bitcast(
                        lax.bitwise_and(w32, himask), jnp.float32)
                return carry2
            lax.fori_loop(0, spans, span_body, 0)
        pltpu.sync_copy(out_v, out_hbm.at[pl.ds(row0, _ROWS)])
        return carry

    lax.fori_loop(c_lo, c_hi, chunk_body, 0)


def _sc_widen(y32):
    n_rows, wpr = y32.shape
    mesh = plsc.VectorSubcoreMesh(core_axis_name="c", subcore_axis_name="s",
                                  num_cores=2, num_subcores=16)
    fn = functools.partial(
        pl.kernel,
        mesh=mesh,
        out_type=jax.ShapeDtypeStruct((n_rows, wpr * 2), jnp.float32),
        scratch_types=[
            pltpu.VMEM((_ROWS, wpr), jnp.int32),
            pltpu.VMEM((_ROWS, wpr * 2), jnp.float32),
        ],
        compiler_params=pltpu.CompilerParams(
            needs_layout_passes=False,
            skip_device_barrier=True,
        ),
    )(_sc_widen_body)
    return fn(y32)


def kernel(input, W1, b1, W2, b2):
    B, N, in_dim = input.shape
    hid = W1.shape[1]
    hist = W2.shape[1] // hid

    y32 = _tc_mlp_bf16(input, W1, b1, W2, b2)
    out = _sc_widen(y32)
    return out.reshape(N, hist, B, hid)


# speedup vs baseline: 44.1330x; 1.1516x over previous
import functools

import jax
import jax.numpy as jnp
from jax import lax
from jax.experimental import pallas as pl
from jax.experimental.pallas import tpu as pltpu
from jax.experimental.pallas import tpu_sc as plsc


def _mlp_kernel(xc_ref, e_ref, b1t_ref, g_ref, b2t_ref, out_ref):
    bh = e_ref.shape[1]
    gw = g_ref.shape[0]
    hist = g_ref.shape[1] // gw
    n_groups = bh // gw

    h = jnp.maximum(
        jnp.dot(xc_ref[...], e_ref[...], preferred_element_type=jnp.float32)
        + b1t_ref[0][None, :],
        0.0,
    )

    rnd = jnp.int32(32768)
    himask = jnp.int32(-65536)

    for e in range(n_groups // 2):
        oa = jnp.dot(h[:, (2 * e) * gw:(2 * e + 1) * gw], g_ref[...],
                     preferred_element_type=jnp.float32)
        ob = jnp.dot(h[:, (2 * e + 1) * gw:(2 * e + 2) * gw], g_ref[...],
                     preferred_element_type=jnp.float32)
        for t in range(hist):
            la = t * bh + (2 * e) * gw
            lb = t * bh + (2 * e + 1) * gw
            a = oa[:, t * gw:(t + 1) * gw] + b2t_ref[0][None, la:la + gw]
            b = ob[:, t * gw:(t + 1) * gw] + b2t_ref[0][None, lb:lb + gw]
            ai = lax.bitcast_convert_type(a, jnp.int32) + rnd
            bi = lax.bitcast_convert_type(b, jnp.int32) + rnd
            w = lax.bitwise_or(
                lax.shift_right_logical(ai, jnp.int32(16)),
                lax.bitwise_and(bi, himask),
            )
            lo = (t * (n_groups // 2) + e) * gw
            out_ref[:, lo:lo + gw] = w


def _tc_mlp_bf16(input, W1, b1, W2, b2):
    B, N, in_dim = input.shape
    hid = W1.shape[1]
    hist = W2.shape[1] // hid

    nb = 400

    xc = jnp.transpose(input, (1, 2, 0)).reshape(N, in_dim * B)
    eye_b = jnp.eye(B, dtype=jnp.float32)
    e_mat = jnp.einsum('bc,dk->dbck', eye_b, W1).reshape(in_dim * B, B * hid)
    b1t = jnp.tile(b1, B).reshape(1, B * hid)
    w2r = W2.reshape(hid, hist, hid)
    eye4 = jnp.eye(4, dtype=jnp.float32)
    g_mat = jnp.einsum('bc,ktj->bktcj', eye4, w2r).reshape(4 * hid,
                                                           hist * 4 * hid)
    b2t = jnp.tile(b2.reshape(hist, 1, hid), (1, B, 1)).reshape(1,
                                                                hist * B * hid)

    return pl.pallas_call(
        _mlp_kernel,
        grid=(N // nb,),
        in_specs=[
            pl.BlockSpec((nb, in_dim * B), lambda i: (i, 0)),
            pl.BlockSpec((in_dim * B, B * hid), lambda i: (0, 0)),
            pl.BlockSpec((1, B * hid), lambda i: (0, 0)),
            pl.BlockSpec((4 * hid, hist * 4 * hid), lambda i: (0, 0)),
            pl.BlockSpec((1, hist * B * hid), lambda i: (0, 0)),
        ],
        out_specs=pl.BlockSpec((nb, hist * B * hid // 2), lambda i: (i, 0)),
        out_shape=jax.ShapeDtypeStruct((N, hist * B * hid // 2), jnp.int32),
        compiler_params=pltpu.CompilerParams(
            dimension_semantics=("parallel",),
        ),
    )(xc, e_mat, b1t, g_mat, b2t)


_NW = 32
_ROWS = 8


def _widen_chunk(in_v, out_v, spans):
    himask = jnp.int32(-65536)
    for r in range(_ROWS):
        def span_body(m, carry, r=r):
            for j in range(8):
                w32 = in_v[r, pl.ds(m * 128 + j * 16, 16)]
                out_v[r, pl.ds(m * 256 + j * 16, 16)] = plsc.bitcast(
                    lax.shift_left(w32, jnp.int32(16)), jnp.float32)
                out_v[r, pl.ds(m * 256 + 128 + j * 16, 16)] = plsc.bitcast(
                    lax.bitwise_and(w32, himask), jnp.float32)
            return carry
        lax.fori_loop(0, spans, span_body, 0)


def _sc_widen_body(in_hbm, out_hbm, in0, in1, out0, out1, si0, si1, so0, so1):
    n_rows = in_hbm.shape[0]
    wpr = in_hbm.shape[1]
    spans = wpr // 128
    n_chunks = n_rows // _ROWS
    wid = lax.axis_index("s") * 2 + lax.axis_index("c")
    c_lo = (wid * n_chunks) // _NW
    c_hi = ((wid + 1) * n_chunks) // _NW
    cc = c_hi - c_lo
    npairs = cc // 2

    def in_cp(c, buf, sem):
        return pltpu.make_async_copy(
            in_hbm.at[pl.ds(c * _ROWS, _ROWS)], buf, sem)

    def out_cp(c, buf, sem):
        return pltpu.make_async_copy(
            buf, out_hbm.at[pl.ds(c * _ROWS, _ROWS)], sem)

    def pair_body(k, carry):
        c0 = c_lo + 2 * k
        c1 = c0 + 1
        in_cp(c0, in0, si0).start()
        in_cp(c1, in1, si1).start()

        @pl.when(k > 0)
        def _():
            out_cp(c0 - 2, out0, so0).wait()
            out_cp(c1 - 2, out1, so1).wait()

        in_cp(c0, in0, si0).wait()
        _widen_chunk(in0, out0, spans)
        out_cp(c0, out0, so0).start()
        in_cp(c1, in1, si1).wait()
        _widen_chunk(in1, out1, spans)
        out_cp(c1, out1, so1).start()
        return carry

    lax.fori_loop(0, npairs, pair_body, 0)
    out_cp(c_lo + 2 * npairs - 2, out0, so0).wait()
    out_cp(c_lo + 2 * npairs - 1, out1, so1).wait()

    @pl.when(cc % 2 == 1)
    def _():
        c = c_lo + 2 * npairs
        pltpu.sync_copy(in_hbm.at[pl.ds(c * _ROWS, _ROWS)], in0)
        _widen_chunk(in0, out0, spans)
        pltpu.sync_copy(out0, out_hbm.at[pl.ds(c * _ROWS, _ROWS)])


def _sc_widen(y32):
    n_rows, wpr = y32.shape
    mesh = plsc.VectorSubcoreMesh(core_axis_name="c", subcore_axis_name="s",
                                  num_cores=2, num_subcores=16)
    fn = functools.partial(
        pl.kernel,
        mesh=mesh,
        out_type=jax.ShapeDtypeStruct((n_rows, wpr * 2), jnp.float32),
        scratch_types=[
            pltpu.VMEM((_ROWS, wpr), jnp.int32),
            pltpu.VMEM((_ROWS, wpr), jnp.int32),
            pltpu.VMEM((_ROWS, wpr * 2), jnp.float32),
            pltpu.VMEM((_ROWS, wpr * 2), jnp.float32),
            pltpu.SemaphoreType.DMA,
            pltpu.SemaphoreType.DMA,
            pltpu.SemaphoreType.DMA,
            pltpu.SemaphoreType.DMA,
        ],
        compiler_params=pltpu.CompilerParams(
            needs_layout_passes=False,
            skip_device_barrier=True,
        ),
    )(_sc_widen_body)
    return fn(y32)


def kernel(input, W1, b1, W2, b2):
    B, N, in_dim = input.shape
    hid = W1.shape[1]
    hist = W2.shape[1] // hid

    y32 = _tc_mlp_bf16(input, W1, b1, W2, b2)
    out = _sc_widen(y32)
    return out.reshape(N, hist, B, hid)


# speedup vs baseline: 93.8413x; 2.1263x over previous
import jax
import jax.numpy as jnp
from jax.experimental import pallas as pl
from jax.experimental.pallas import tpu as pltpu


def _mlp_kernel(xc_ref, e_ref, b1t_ref, g_ref, b2t_ref, out_ref):
    bh = e_ref.shape[1]
    gw = g_ref.shape[0]
    hist_gw = g_ref.shape[1]
    n_groups = bh // gw

    h = jnp.maximum(
        jnp.dot(xc_ref[...], e_ref[...], preferred_element_type=jnp.float32)
        + b1t_ref[0][None, :],
        0.0,
    )

    hist = hist_gw // gw
    for g in range(n_groups):
        og = jnp.dot(h[:, g * gw:(g + 1) * gw], g_ref[...],
                     preferred_element_type=jnp.float32)
        for t in range(hist):
            lo = t * bh + g * gw
            out_ref[:, lo:lo + gw] = (
                og[:, t * gw:(t + 1) * gw] + b2t_ref[0][None, lo:lo + gw]
            )


def kernel(input, W1, b1, W2, b2):
    B, N, in_dim = input.shape
    hid = W1.shape[1]
    hist = W2.shape[1] // hid

    nb = 400

    xc = jnp.transpose(input, (1, 2, 0)).reshape(N, in_dim * B)
    eye_b = jnp.eye(B, dtype=jnp.float32)
    e_mat = jnp.einsum('bc,dk->dbck', eye_b, W1).reshape(in_dim * B, B * hid)
    b1t = jnp.tile(b1, B).reshape(1, B * hid)
    w2r = W2.reshape(hid, hist, hid)
    eye4 = jnp.eye(4, dtype=jnp.float32)
    g_mat = jnp.einsum('bc,ktj->bktcj', eye4, w2r).reshape(4 * hid,
                                                           hist * 4 * hid)
    b2t = jnp.tile(b2.reshape(hist, 1, hid), (1, B, 1)).reshape(1,
                                                                hist * B * hid)

    out = pl.pallas_call(
        _mlp_kernel,
        grid=(N // nb,),
        in_specs=[
            pl.BlockSpec((nb, in_dim * B), lambda i: (i, 0)),
            pl.BlockSpec((in_dim * B, B * hid), lambda i: (0, 0)),
            pl.BlockSpec((1, B * hid), lambda i: (0, 0)),
            pl.BlockSpec((4 * hid, hist * 4 * hid), lambda i: (0, 0)),
            pl.BlockSpec((1, hist * B * hid), lambda i: (0, 0)),
        ],
        out_specs=pl.BlockSpec((nb, hist * B * hid), lambda i: (i, 0)),
        out_shape=jax.ShapeDtypeStruct((N, hist * B * hid), jnp.float32),
        compiler_params=pltpu.CompilerParams(
            dimension_semantics=("parallel",),
        ),
    )(xc, e_mat, b1t, g_mat, b2t)
    return out.reshape(N, hist, B, hid)
